# baseline (speedup 1.0000x reference)
import functools

import jax
import jax.numpy as jnp
from jax import lax
from jax.experimental import pallas as pl
from jax.experimental.pallas import tpu as pltpu
from jax.experimental.pallas import tpu_sc as plsc

N_FACTORS = 64
BATCH = 16384
NC = 2
NS = 16
NW = NC * NS
B_PER_W = BATCH // NW
IDX_CHUNK = 128
N_CHUNKS = B_PER_W // IDX_CHUNK
GROUPS = B_PER_W // 16


def _sc_body(users_hbm, items_hbm, uf_hbm, if_hbm, out_hbm,
             idx_u, idx_v, u_rows, v_rows, out_buf, sem):
    wid = lax.axis_index("s") * NC + lax.axis_index("c")

    pltpu.sync_copy(users_hbm.at[pl.ds(wid * N_CHUNKS, N_CHUNKS)], idx_u)
    pltpu.sync_copy(items_hbm.at[pl.ds(wid * N_CHUNKS, N_CHUNKS)], idx_v)

    copies = []
    for j in range(N_CHUNKS):
        dst = u_rows.at[pl.ds(j * IDX_CHUNK, IDX_CHUNK)]
        copies.append(pltpu.async_copy(uf_hbm.at[idx_u.at[j]], dst, sem))
    for j in range(N_CHUNKS):
        dst = v_rows.at[pl.ds(j * IDX_CHUNK, IDX_CHUNK)]
        copies.append(pltpu.async_copy(if_hbm.at[idx_v.at[j]], dst, sem))
    for c in copies:
        c.wait()

    lane = lax.iota(jnp.int32, 16)

    def group_body(g, _):
        rows = g * 16 + lane
        acc0 = jnp.zeros((16,), jnp.float32)
        acc1 = jnp.zeros((16,), jnp.float32)
        acc2 = jnp.zeros((16,), jnp.float32)
        acc3 = jnp.zeros((16,), jnp.float32)
        accs = [acc0, acc1, acc2, acc3]
        for d in range(N_FACTORS):
            col = jnp.full((16,), d, jnp.int32)
            u = plsc.load_gather(u_rows, [rows, col])
            v = plsc.load_gather(v_rows, [rows, col])
            accs[d % 4] = accs[d % 4] + u * v
        out_buf[g] = (accs[0] + accs[1]) + (accs[2] + accs[3])
        return 0

    lax.fori_loop(0, GROUPS, group_body, 0)

    pltpu.sync_copy(out_buf, out_hbm.at[pl.ds(wid * GROUPS, GROUPS)])


@jax.jit
def _mf_dot(users, items, user_factors, item_factors):
    mesh = plsc.VectorSubcoreMesh(
        core_axis_name="c", subcore_axis_name="s",
        num_cores=NC, num_subcores=NS)
    k = pl.kernel(
        _sc_body,
        out_type=jax.ShapeDtypeStruct((BATCH // 16, 16), jnp.float32),
        mesh=mesh,
        compiler_params=pltpu.CompilerParams(
            needs_layout_passes=False, use_tc_tiling_on_sc=False),
        scratch_types=[
            pltpu.VMEM((N_CHUNKS, IDX_CHUNK), jnp.int32),
            pltpu.VMEM((N_CHUNKS, IDX_CHUNK), jnp.int32),
            pltpu.VMEM((B_PER_W, N_FACTORS), jnp.float32),
            pltpu.VMEM((B_PER_W, N_FACTORS), jnp.float32),
            pltpu.VMEM((GROUPS, 16), jnp.float32),
            pltpu.SemaphoreType.DMA,
        ],
    )
    return k(users, items, user_factors, item_factors)


def kernel(data, user_factors, item_factors):
    users = data[:, 0].astype(jnp.int32).reshape(NW * N_CHUNKS, IDX_CHUNK)
    items = data[:, 1].astype(jnp.int32).reshape(NW * N_CHUNKS, IDX_CHUNK)
    out = _mf_dot(users, items, user_factors, item_factors)
    return out.reshape(BATCH)
